# baseline (speedup 1.0000x reference)
import functools

import jax
import jax.numpy as jnp
from jax import lax
from jax.experimental import pallas as pl
from jax.experimental.pallas import tpu as pltpu
from jax.experimental.pallas import tpu_sc as plsc

NBUF = 2
K = 8


@functools.cache
def _make_kernel(V, D, B):
    info = plsc.get_sparse_core_info()
    NC, NS = info.num_cores, info.num_subcores
    NW = NC * NS
    assert B % NW == 0
    b_per_w = B // NW
    C = 2048
    CS = C // K
    assert b_per_w % (C * NBUF) == 0
    n_outer = b_per_w // (C * NBUF)
    mesh = plsc.VectorSubcoreMesh(core_axis_name="c", subcore_axis_name="s")

    @functools.partial(
        pl.kernel,
        out_type=jax.ShapeDtypeStruct((B, D), jnp.float32),
        mesh=mesh,
        scratch_types=[
            pltpu.VMEM((NBUF, C), jnp.int32),
            pltpu.VMEM((NBUF, C, D), jnp.float32),
            [pltpu.SemaphoreType.DMA] * NBUF,
            [pltpu.SemaphoreType.DMA] * NBUF,
            [pltpu.SemaphoreType.DMA] * NBUF,
        ],
        compiler_params=pltpu.CompilerParams(use_tc_tiling_on_sc=False),
    )
    def k(x_hbm, table_hbm, out_hbm, idx_v, rows_v, sem_i, sem_g, sem_s):
        wid = lax.axis_index("s") * NC + lax.axis_index("c")
        base = wid * b_per_w

        for b in range(NBUF):
            pltpu.async_copy(
                x_hbm.at[pl.ds(base + b * C, C)], idx_v.at[b], sem_i[b]
            )

        def outer(j, carry):
            for b in range(NBUF):
                off = base + (j * NBUF + b) * C
                pltpu.make_async_copy(
                    x_hbm.at[pl.ds(off, C)], idx_v.at[b], sem_i[b]
                ).wait()
                @pl.when(j > 0)
                def _():
                    pltpu.make_async_copy(
                        rows_v.at[b], out_hbm.at[pl.ds(base, C)], sem_s[b]
                    ).wait()
                for kk in range(K):
                    pltpu.async_copy(
                        table_hbm.at[idx_v.at[b].at[pl.ds(kk * CS, CS)]],
                        rows_v.at[b].at[pl.ds(kk * CS, CS)],
                        sem_g[b],
                    )
                pltpu.make_async_copy(
                    rows_v.at[b], out_hbm.at[pl.ds(base, C)], sem_g[b]
                ).wait()
                pltpu.async_copy(
                    rows_v.at[b], out_hbm.at[pl.ds(off, C)], sem_s[b]
                )
                @pl.when(j < n_outer - 1)
                def _():
                    nxt = off + NBUF * C
                    pltpu.async_copy(
                        x_hbm.at[pl.ds(nxt, C)], idx_v.at[b], sem_i[b]
                    )
            return carry

        lax.fori_loop(0, n_outer, outer, 0)

        for b in range(NBUF):
            pltpu.make_async_copy(
                rows_v.at[b], out_hbm.at[pl.ds(base, C)], sem_s[b]
            ).wait()

    return k


def kernel(x, table):
    B = x.shape[0] * x.shape[1]
    V, D = table.shape
    xf = x.reshape(B).astype(jnp.int32)
    xf = jax.lax.optimization_barrier(xf)
    tbl_flat = jax.lax.optimization_barrier(table.reshape(V * D))
    tbl_lin = tbl_flat.reshape(V, D)
    out = _make_kernel(V, D, B)(xf, tbl_lin)
    return out.reshape(x.shape[0], x.shape[1], D)


# speedup vs baseline: 1.3685x; 1.3685x over previous
import functools

import jax
import jax.numpy as jnp
from jax import lax
from jax.experimental import pallas as pl
from jax.experimental.pallas import tpu as pltpu
from jax.experimental.pallas import tpu_sc as plsc

NBUF = 2
UNROLL = 8


@functools.cache
def _make_kernel(V, D, NB, H):
    info = plsc.get_sparse_core_info()
    NC, NS = info.num_cores, info.num_subcores
    NW = NC * NS
    assert NB % NW == 0 and H % NBUF == 0
    Cb = NB // NW
    mesh = plsc.VectorSubcoreMesh(core_axis_name="c", subcore_axis_name="s")

    @functools.partial(
        pl.kernel,
        out_type=jax.ShapeDtypeStruct((H, D, NB), jnp.float32),
        mesh=mesh,
        scratch_types=[
            pltpu.VMEM((NBUF, Cb), jnp.int32),
            pltpu.VMEM((NBUF, Cb, D), jnp.float32),
            pltpu.VMEM((NBUF, D, Cb), jnp.float32),
            [pltpu.SemaphoreType.DMA] * NBUF,
            [pltpu.SemaphoreType.DMA] * NBUF,
            [pltpu.SemaphoreType.DMA] * NBUF,
        ],
        compiler_params=pltpu.CompilerParams(
            use_tc_tiling_on_sc=False, needs_layout_passes=False
        ),
    )
    def k(xt_hbm, table_hbm, out_hbm, idx_v, rows_v, trows_v,
          sem_i, sem_g, sem_s):
        wid = lax.axis_index("s") * NC + lax.axis_index("c")
        b0 = wid * Cb
        lane = jnp.arange(D, dtype=jnp.int32)

        for b in range(NBUF):
            pltpu.async_copy(
                xt_hbm.at[pl.ds(b * NB + b0, Cb)], idx_v.at[b], sem_i[b]
            )
        pltpu.make_async_copy(
            xt_hbm.at[pl.ds(0, Cb)], idx_v.at[0], sem_i[0]
        ).wait()
        pltpu.async_copy(table_hbm.at[idx_v.at[0]], rows_v.at[0], sem_g[0])

        def outer(j, carry):
            for b in range(NBUF):
                h = j * NBUF + b
                nb = (b + 1) % NBUF
                @pl.when(h + 1 < H)
                def _():
                    pltpu.make_async_copy(
                        xt_hbm.at[pl.ds(0, Cb)], idx_v.at[nb], sem_i[nb]
                    ).wait()
                    pltpu.async_copy(
                        table_hbm.at[idx_v.at[nb]], rows_v.at[nb], sem_g[nb]
                    )
                pltpu.make_async_copy(
                    table_hbm.at[idx_v.at[b]], rows_v.at[b], sem_g[b]
                ).wait()
                @pl.when(j > 0)
                def _():
                    pltpu.make_async_copy(
                        table_hbm.at[pl.ds(0, Cb)], rows_v.at[b], sem_s[b]
                    ).wait()

                def transp(j0, c):
                    for u in range(UNROLL):
                        jj = j0 * UNROLL + u
                        row = rows_v.at[b][jj]
                        plsc.store_scatter(
                            trows_v.at[b], [lane, jnp.full((D,), jj,
                                                           jnp.int32)], row
                        )
                    return c

                lax.fori_loop(0, Cb // UNROLL, transp, 0)

                pltpu.async_copy(
                    trows_v.at[b],
                    out_hbm.at[h].at[pl.ds(0, D), pl.ds(b0, Cb)],
                    sem_s[b],
                )
                @pl.when(h + NBUF < H)
                def _():
                    pltpu.async_copy(
                        xt_hbm.at[pl.ds((h + NBUF) * NB + b0, Cb)],
                        idx_v.at[b],
                        sem_i[b],
                    )
            return carry

        lax.fori_loop(0, H // NBUF, outer, 0)

        for b in range(NBUF):
            pltpu.make_async_copy(
                table_hbm.at[pl.ds(0, Cb)], rows_v.at[b], sem_s[b]
            ).wait()

    return k


def kernel(x, table):
    NB, H = x.shape
    V, D = table.shape
    xt = jnp.transpose(x, (1, 0)).astype(jnp.int32).reshape(H * NB)
    tbl_t = jax.lax.optimization_barrier(jnp.transpose(table, (1, 0)))
    tbl_lin = jnp.transpose(tbl_t, (1, 0))
    out_t = _make_kernel(V, D, NB, H)(xt, tbl_lin)
    return jnp.transpose(out_t, (2, 0, 1))


# speedup vs baseline: 2.0915x; 1.5282x over previous
import functools

import jax
import jax.numpy as jnp
from jax import lax
from jax.experimental import pallas as pl
from jax.experimental.pallas import tpu as pltpu
from jax.experimental.pallas import tpu_sc as plsc

NBUF = 2
UNROLL = 8


@functools.cache
def _make_kernel(V, D, NB, H):
    info = plsc.get_sparse_core_info()
    NC, NS = info.num_cores, info.num_subcores
    NW = NC * NS
    assert NB % NW == 0 and H % NBUF == 0
    Cb = NB // NW
    mesh = plsc.VectorSubcoreMesh(core_axis_name="c", subcore_axis_name="s")

    @functools.partial(
        pl.kernel,
        out_type=jax.ShapeDtypeStruct((H, D, NB), jnp.float32),
        mesh=mesh,
        scratch_types=[
            pltpu.VMEM((NBUF, Cb), jnp.int32),
            pltpu.VMEM((NBUF, Cb, D), jnp.float32),
            pltpu.VMEM((NBUF, D, Cb + 1), jnp.float32),
            [pltpu.SemaphoreType.DMA] * NBUF,
            [pltpu.SemaphoreType.DMA] * NBUF,
            [pltpu.SemaphoreType.DMA] * NBUF,
        ],
        compiler_params=pltpu.CompilerParams(
            use_tc_tiling_on_sc=False, needs_layout_passes=False
        ),
    )
    def k(xt_hbm, table_hbm, out_hbm, idx_v, rows_v, trows_v,
          sem_i, sem_g, sem_s):
        wid = lax.axis_index("s") * NC + lax.axis_index("c")
        b0 = wid * Cb
        lane = jnp.arange(D, dtype=jnp.int32)

        for b in range(NBUF):
            pltpu.async_copy(
                xt_hbm.at[pl.ds(b * NB + b0, Cb)], idx_v.at[b], sem_i[b]
            )
        pltpu.make_async_copy(
            xt_hbm.at[pl.ds(0, Cb)], idx_v.at[0], sem_i[0]
        ).wait()
        pltpu.async_copy(table_hbm.at[idx_v.at[0]], rows_v.at[0], sem_g[0])

        def outer(j, carry):
            for b in range(NBUF):
                h = j * NBUF + b
                nb = (b + 1) % NBUF
                @pl.when(h + 1 < H)
                def _():
                    pltpu.make_async_copy(
                        xt_hbm.at[pl.ds(0, Cb)], idx_v.at[nb], sem_i[nb]
                    ).wait()
                    pltpu.async_copy(
                        table_hbm.at[idx_v.at[nb]], rows_v.at[nb], sem_g[nb]
                    )
                pltpu.make_async_copy(
                    table_hbm.at[idx_v.at[b]], rows_v.at[b], sem_g[b]
                ).wait()
                @pl.when(j > 0)
                def _():
                    pltpu.make_async_copy(
                        table_hbm.at[pl.ds(0, Cb)], rows_v.at[b], sem_s[b]
                    ).wait()

                def transp(j0, c):
                    for u in range(UNROLL):
                        jj = j0 * UNROLL + u
                        row = rows_v.at[b][jj]
                        plsc.store_scatter(
                            trows_v.at[b], [lane, jnp.full((D,), jj,
                                                           jnp.int32)], row
                        )
                    return c

                lax.fori_loop(0, Cb // UNROLL, transp, 0)

                pltpu.async_copy(
                    trows_v.at[b].at[pl.ds(0, D), pl.ds(0, Cb)],
                    out_hbm.at[h].at[pl.ds(0, D), pl.ds(b0, Cb)],
                    sem_s[b],
                )
                @pl.when(h + NBUF < H)
                def _():
                    pltpu.async_copy(
                        xt_hbm.at[pl.ds((h + NBUF) * NB + b0, Cb)],
                        idx_v.at[b],
                        sem_i[b],
                    )
            return carry

        lax.fori_loop(0, H // NBUF, outer, 0)

        for b in range(NBUF):
            pltpu.make_async_copy(
                table_hbm.at[pl.ds(0, Cb)], rows_v.at[b], sem_s[b]
            ).wait()

    return k


def kernel(x, table):
    NB, H = x.shape
    V, D = table.shape
    xt = jnp.transpose(x, (1, 0)).astype(jnp.int32).reshape(H * NB)
    tbl_t = jax.lax.optimization_barrier(jnp.transpose(table, (1, 0)))
    tbl_lin = jnp.transpose(tbl_t, (1, 0))
    out_t = _make_kernel(V, D, NB, H)(xt, tbl_lin)
    return jnp.transpose(out_t, (2, 0, 1))


# speedup vs baseline: 2.4808x; 1.1862x over previous
import functools

import jax
import jax.numpy as jnp
from jax import lax
from jax.experimental import pallas as pl
from jax.experimental.pallas import tpu as pltpu
from jax.experimental.pallas import tpu_sc as plsc

NBUF = 2
UNROLL = 8


@functools.cache
def _make_kernel(V, D, NB, H):
    info = plsc.get_sparse_core_info()
    NC, NS = info.num_cores, info.num_subcores
    NW = NC * NS
    assert NB % NW == 0 and H % NBUF == 0
    Cb = NB // NW
    mesh = plsc.VectorSubcoreMesh(core_axis_name="c", subcore_axis_name="s")

    assert D % 8 == 0 and NB % 128 == 0 and Cb % 128 == 0

    @functools.partial(
        pl.kernel,
        out_type=jax.ShapeDtypeStruct((H, D // 8, NB // 128, 8, 128),
                                      jnp.float32),
        mesh=mesh,
        scratch_types=[
            pltpu.VMEM((NBUF, Cb), jnp.int32),
            pltpu.VMEM((NBUF, Cb, D), jnp.float32),
            pltpu.VMEM((NBUF, D, Cb + 1), jnp.float32),
            [pltpu.SemaphoreType.DMA] * NBUF,
            [pltpu.SemaphoreType.DMA] * NBUF,
            [pltpu.SemaphoreType.DMA] * NBUF,
        ],
        compiler_params=pltpu.CompilerParams(
            use_tc_tiling_on_sc=False, needs_layout_passes=False
        ),
    )
    def k(xt_hbm, table_hbm, out_hbm, idx_v, rows_v, trows_v,
          sem_i, sem_g, sem_s):
        wid = lax.axis_index("s") * NC + lax.axis_index("c")
        b0 = wid * Cb
        lane = jnp.arange(D, dtype=jnp.int32)

        for b in range(NBUF):
            pltpu.async_copy(
                xt_hbm.at[pl.ds(b * NB + b0, Cb)], idx_v.at[b], sem_i[b]
            )
        pltpu.make_async_copy(
            xt_hbm.at[pl.ds(0, Cb)], idx_v.at[0], sem_i[0]
        ).wait()
        pltpu.async_copy(table_hbm.at[idx_v.at[0]], rows_v.at[0], sem_g[0])

        def outer(j, carry):
            for b in range(NBUF):
                h = j * NBUF + b
                nb = (b + 1) % NBUF
                @pl.when(h + 1 < H)
                def _():
                    pltpu.make_async_copy(
                        xt_hbm.at[pl.ds(0, Cb)], idx_v.at[nb], sem_i[nb]
                    ).wait()
                    pltpu.async_copy(
                        table_hbm.at[idx_v.at[nb]], rows_v.at[nb], sem_g[nb]
                    )
                pltpu.make_async_copy(
                    table_hbm.at[idx_v.at[b]], rows_v.at[b], sem_g[b]
                ).wait()
                @pl.when(j > 0)
                def _():
                    pltpu.make_async_copy(
                        table_hbm.at[pl.ds(0, Cb)], rows_v.at[b], sem_s[b]
                    ).wait()

                def transp(j0, c):
                    for u in range(UNROLL):
                        jj = j0 * UNROLL + u
                        row = rows_v.at[b][jj]
                        plsc.store_scatter(
                            trows_v.at[b], [lane, jnp.full((D,), jj,
                                                           jnp.int32)], row
                        )
                    return c

                lax.fori_loop(0, Cb // UNROLL, transp, 0)

                for s in range(D // 8):
                    for t in range(Cb // 128):
                        pltpu.async_copy(
                            trows_v.at[b].at[pl.ds(8 * s, 8),
                                             pl.ds(128 * t, 128)],
                            out_hbm.at[h].at[s].at[b0 // 128 + t],
                            sem_s[b],
                        )
                @pl.when(h + NBUF < H)
                def _():
                    pltpu.async_copy(
                        xt_hbm.at[pl.ds((h + NBUF) * NB + b0, Cb)],
                        idx_v.at[b],
                        sem_i[b],
                    )
            return carry

        lax.fori_loop(0, H // NBUF, outer, 0)

        for b in range(NBUF):
            pltpu.make_async_copy(
                table_hbm.at[pl.ds(0, Cb)], rows_v.at[b], sem_s[b]
            ).wait()

    return k


def kernel(x, table):
    NB, H = x.shape
    V, D = table.shape
    xt = jnp.transpose(x, (1, 0)).astype(jnp.int32).reshape(H * NB)
    tbl_t = jax.lax.optimization_barrier(jnp.transpose(table, (1, 0)))
    tbl_lin = jnp.transpose(tbl_t, (1, 0))
    out5 = _make_kernel(V, D, NB, H)(xt, tbl_lin)
    return jnp.transpose(out5, (2, 4, 0, 1, 3)).reshape(NB, H, D)
